# baseline (speedup 1.0000x reference)
import jax
import jax.numpy as jnp
from jax import lax
from jax.experimental import pallas as pl
from jax.experimental.pallas import tpu as pltpu
from jax.experimental.pallas import tpu_sc as plsc

N = 10000
E = 320000
D = 128
DE = 16
DT = 16

NC = 2
NS = 16
NW = NC * NS
NH = 2
EH = E // NH
EW = EH // NW
C = 40
NCHUNK = EW // C
NP = 10240
ROWS_PT = NP // NS

BLK_E = 2000
BLK_N = 2000

_MESH = plsc.VectorSubcoreMesh(
    core_axis_name="c", subcore_axis_name="s", num_cores=NC, num_subcores=NS
)


def _sp(x):
    return jax.nn.softplus(x)


def _dot_t(x, w):
    return lax.dot_general(
        x, w, (((1,), (1,)), ((), ())), preferred_element_type=jnp.float32
    )


def _pack_bf16_pair(x):
    u = lax.bitcast_convert_type(x.astype(jnp.bfloat16), jnp.uint16)
    k = u.shape[1] // 2
    lo = u[:, :k].astype(jnp.uint32)
    hi = u[:, k:].astype(jnp.uint32)
    return lax.bitcast_convert_type(lo | (hi << 16), jnp.int32)


def _unpack_bf16_pair(w):
    f_lo = lax.bitcast_convert_type(lax.shift_left(w, 16), jnp.float32)
    f_hi = lax.bitcast_convert_type(
        jnp.bitwise_and(w, jnp.int32(-65536)), jnp.float32)
    return jnp.concatenate([f_lo, f_hi], axis=1)


def _pack2(x):
    hi = _pack_bf16_pair(x)
    lo = _pack_bf16_pair(x - _unpack_bf16_pair(hi))
    return jnp.concatenate([hi, lo], axis=1)


def _unpack2(w):
    k = w.shape[1] // 2
    return _unpack_bf16_pair(w[:, :k]) + _unpack_bf16_pair(w[:, k:])


def _gather_body(hh_hbm, c3_hbm, hs_hbm, hhr_hbm,
                 cidx, bg0, bg1, gs0, gs1, ws0, ws1):
    ci = lax.axis_index("c")
    si = lax.axis_index("s")
    wid = si * NC + ci
    pltpu.sync_copy(c3_hbm.at[wid], cidx)

    def fire_gather(j, bg, sem):
        pltpu.async_copy(hh_hbm.at[cidx.at[j]], bg, sem)

    def wait_gather(j, bg, sem):
        pltpu.make_async_copy(hh_hbm.at[cidx.at[j]], bg, sem).wait()

    def fire_write(j, bg, sem):
        rows = wid * EW + j * C
        pltpu.async_copy(bg.at[pl.ds(0, C)], hs_hbm.at[pl.ds(rows, C)], sem)
        pltpu.async_copy(bg.at[pl.ds(C, C)], hhr_hbm.at[pl.ds(rows, C)], sem)

    def wait_write(j, bg, sem):
        rows = wid * EW + j * C
        pltpu.make_async_copy(
            bg.at[pl.ds(0, C)], hs_hbm.at[pl.ds(rows, C)], sem).wait()
        pltpu.make_async_copy(
            bg.at[pl.ds(C, C)], hhr_hbm.at[pl.ds(rows, C)], sem).wait()

    fire_gather(0, bg0, gs0)

    def body(t, carry):
        j0 = 2 * t
        fire_gather(j0 + 1, bg1, gs1)
        wait_gather(j0, bg0, gs0)
        fire_write(j0, bg0, ws0)
        wait_gather(j0 + 1, bg1, gs1)
        fire_write(j0 + 1, bg1, ws1)
        wait_write(j0, bg0, ws0)
        fire_gather(j0 + 2, bg0, gs0)
        wait_write(j0 + 1, bg1, ws1)
        return carry

    lax.fori_loop(0, (NCHUNK - 1) // 2, body, 0)
    last = NCHUNK - 1
    wait_gather(last, bg0, gs0)
    fire_write(last, bg0, ws0)
    wait_write(last, bg0, ws0)


_gather_call = pl.kernel(
    _gather_body,
    out_type=(
        jax.ShapeDtypeStruct((EH, D), jnp.int32),
        jax.ShapeDtypeStruct((EH, D), jnp.int32),
    ),
    mesh=_MESH,
    scratch_types=(
        pltpu.VMEM((NCHUNK, 2 * C), jnp.int32),
        pltpu.VMEM((2 * C, D), jnp.int32),
        pltpu.VMEM((2 * C, D), jnp.int32),
        pltpu.SemaphoreType.DMA,
        pltpu.SemaphoreType.DMA,
        pltpu.SemaphoreType.DMA,
        pltpu.SemaphoreType.DMA,
    ),
)


def _scatter_body(msg_hbm, r3_hbm, z_hbm, p_hbm, ridx, b0, b1, acc, s0, s1):
    ci = lax.axis_index("c")
    si = lax.axis_index("s")
    wid = si * NC + ci
    pltpu.sync_copy(z_hbm.at[pl.ds(si * ROWS_PT, ROWS_PT)],
                    acc.at[pl.ds(si * ROWS_PT, ROWS_PT)])
    pltpu.sync_copy(r3_hbm.at[wid], ridx)
    plsc.subcore_barrier()

    def fire_load(t, buf, sem):
        rows = wid * EW + t * (2 * C)
        pltpu.async_copy(msg_hbm.at[pl.ds(rows, 2 * C)], buf, sem)

    def wait_load(t, buf, sem):
        rows = wid * EW + t * (2 * C)
        pltpu.make_async_copy(
            msg_hbm.at[pl.ds(rows, 2 * C)], buf, sem).wait()

    def add2(t, buf):
        pltpu.sync_copy(buf.at[pl.ds(0, C)], acc.at[ridx.at[2 * t]],
                        add=True)
        pltpu.sync_copy(buf.at[pl.ds(C, C)], acc.at[ridx.at[2 * t + 1]],
                        add=True)

    npair = NCHUNK // 2
    fire_load(0, b0, s0)

    def body(t, carry):
        t0 = 2 * t
        fire_load(t0 + 1, b1, s1)
        wait_load(t0, b0, s0)
        add2(t0, b0)
        fire_load(t0 + 2, b0, s0)
        wait_load(t0 + 1, b1, s1)
        add2(t0 + 1, b1)
        return carry

    lax.fori_loop(0, npair // 2 - 1, body, 0)
    fire_load(npair - 1, b1, s1)
    wait_load(npair - 2, b0, s0)
    add2(npair - 2, b0)
    wait_load(npair - 1, b1, s1)
    add2(npair - 1, b1)
    last = NCHUNK - 1
    rows = wid * EW + last * C
    pltpu.sync_copy(msg_hbm.at[pl.ds(rows, C)], b0.at[pl.ds(0, C)])
    pltpu.sync_copy(b0.at[pl.ds(0, C)], acc.at[ridx.at[last]], add=True)
    plsc.subcore_barrier()
    pltpu.sync_copy(acc.at[pl.ds(si * ROWS_PT, ROWS_PT)],
                    p_hbm.at[ci, pl.ds(si * ROWS_PT, ROWS_PT)])


_scatter_call = pl.kernel(
    _scatter_body,
    out_type=jax.ShapeDtypeStruct((NC, NP, D), jnp.float32),
    mesh=_MESH,
    scratch_types=(
        pltpu.VMEM((NCHUNK, C), jnp.int32),
        pltpu.VMEM((2 * C, D), jnp.float32),
        pltpu.VMEM((2 * C, D), jnp.float32),
        pltpu.VMEM_SHARED((NP, D), jnp.float32),
        pltpu.SemaphoreType.DMA,
        pltpu.SemaphoreType.DMA,
    ),
)


def _full_spec(a):
    nd = a.ndim
    return pl.BlockSpec(a.shape, lambda i, _nd=nd: (0,) * _nd)


def _init_body(x_ref, faW, fab, h_ref, hh_ref):
    h = _dot_t(x_ref[...], faW[...]) + fab[...]
    h_ref[...] = h
    hh_ref[...] = _pack2(h)


def _init_call(x, faW, fab):
    row = pl.BlockSpec((BLK_N, D), lambda i: (i, 0))
    return pl.pallas_call(
        _init_body,
        grid=(N // BLK_N,),
        in_specs=[row] + [_full_spec(a) for a in (faW, fab)],
        out_specs=[row, row],
        out_shape=[
            jax.ShapeDtypeStruct((N, D), jnp.float32),
            jax.ShapeDtypeStruct((N, D), jnp.int32),
        ],
    )(x, faW, fab)


def _edge_body_first(hs_ref, hhr_ref, ea_ref, fbW, fbb,
                     feW1, feb1, feW2, feb2, Wh, fvb1, We, fvW2, fvb2,
                     eout_ref, msg_ref):
    e_in = _dot_t(ea_ref[...], fbW[...]) + fbb[...]
    _edge_core(hs_ref, hhr_ref, e_in,
               feW1, feb1, feW2, feb2, Wh, fvb1, We, fvW2, fvb2,
               eout_ref, msg_ref)


def _edge_body_rest(hs_ref, hhr_ref, ein_ref,
                    feW1, feb1, feW2, feb2, Wh, fvb1, We, fvW2, fvb2,
                    eout_ref, msg_ref):
    _edge_core(hs_ref, hhr_ref, ein_ref[...],
               feW1, feb1, feW2, feb2, Wh, fvb1, We, fvW2, fvb2,
               eout_ref, msg_ref)


def _edge_core(hs_ref, hhr_ref, e_in,
               feW1, feb1, feW2, feb2, Wh, fvb1, We, fvW2, fvb2,
               eout_ref, msg_ref):
    hs = _unpack2(hs_ref[...])
    hrr = _unpack2(hhr_ref[...])
    hpr = _dot_t(hrr, Wh[...]) + fvb1[...]
    c2 = hs * hrr
    he = _sp(_dot_t(c2, feW1[...]) + feb1[...])
    e_new = _dot_t(he, feW2[...]) + feb2[...] + e_in
    hv = _sp(hpr + _dot_t(e_new, We[...]))
    msg = _dot_t(hv, fvW2[...]) + fvb2[...]
    eout_ref[...] = e_new
    msg_ref[...] = msg


def _edge_step(hs, hhr, ein, fbW, fbb,
               feW1, feb1, feW2, feb2, Wh, fvb1, We, fvW2, fvb2, first):
    row = pl.BlockSpec((BLK_E, D), lambda i: (i, 0))
    irow = pl.BlockSpec((BLK_E, D), lambda i: (i, 0))
    erow = pl.BlockSpec((BLK_E, DE), lambda i: (i, 0))
    if first:
        body = _edge_body_first
        winputs = (fbW, fbb, feW1, feb1, feW2, feb2, Wh, fvb1, We,
                   fvW2, fvb2)
    else:
        body = _edge_body_rest
        winputs = (feW1, feb1, feW2, feb2, Wh, fvb1, We, fvW2, fvb2)
    return pl.pallas_call(
        body,
        grid=(EH // BLK_E,),
        in_specs=[irow, irow, erow] + [_full_spec(a) for a in winputs],
        out_specs=[erow, row],
        out_shape=[
            jax.ShapeDtypeStruct((EH, DE), jnp.float32),
            jax.ShapeDtypeStruct((EH, D), jnp.float32),
        ],
    )(hs, hhr, ein, *winputs)


def _node_mid_body(h_ref, pa_ref, pb_ref, h_out, hh_out):
    hn = h_ref[...] + (pa_ref[0] + pa_ref[1]) + (pb_ref[0] + pb_ref[1])
    h_out[...] = hn
    hh_out[...] = _pack2(hn)


def _node_mid_call(h, pa, pb):
    row = pl.BlockSpec((BLK_N, D), lambda i: (i, 0))
    prow = pl.BlockSpec((NC, BLK_N, D), lambda i: (0, i, 0))
    return pl.pallas_call(
        _node_mid_body,
        grid=(N // BLK_N,),
        in_specs=[row, prow, prow],
        out_specs=[row, row],
        out_shape=[
            jax.ShapeDtypeStruct((N, D), jnp.float32),
            jax.ShapeDtypeStruct((N, D), jnp.int32),
        ],
    )(h, pa, pb)


def _node_fin_body(h_ref, pa_ref, pb_ref, nt_ref,
                   m1W1, m1b1, m1W2, m1b2,
                   m2W1, m2b1, m2W2, m2b2, m2W3, m2b3,
                   force_ref, g_ref):
    hn = h_ref[...] + (pa_ref[0] + pa_ref[1]) + (pb_ref[0] + pb_ref[1])
    t = _sp(_dot_t(hn, m1W1[...]) + m1b1[...])
    force_ref[...] = _dot_t(t, m1W2[...]) + m1b2[...]
    g = _sp(_dot_t(nt_ref[...], m2W1[...]) + m2b1[...])
    g = _sp(_dot_t(g, m2W2[...]) + m2b2[...])
    g_ref[...] = _sp(_dot_t(g, m2W3[...]) + m2b3[...])


def _node_fin_call(h, pa, pb, nt, m1W1, m1b1, m1W2, m1b2,
                   m2W1, m2b1, m2W2, m2b2, m2W3, m2b3):
    row = pl.BlockSpec((BLK_N, D), lambda i: (i, 0))
    prow = pl.BlockSpec((NC, BLK_N, D), lambda i: (0, i, 0))
    ntrow = pl.BlockSpec((BLK_N, DT), lambda i: (i, 0))
    ws = (m1W1, m1b1, m1W2, m1b2, m2W1, m2b1, m2W2, m2b2, m2W3, m2b3)
    return pl.pallas_call(
        _node_fin_body,
        grid=(N // BLK_N,),
        in_specs=[row, prow, prow, ntrow] + [_full_spec(a) for a in ws],
        out_specs=[
            pl.BlockSpec((BLK_N, 8), lambda i: (i, 0)),
            pl.BlockSpec((BLK_N, 16), lambda i: (i, 0)),
        ],
        out_shape=[
            jax.ShapeDtypeStruct((N, 8), jnp.float32),
            jax.ShapeDtypeStruct((N, 16), jnp.float32),
        ],
    )(h, pa, pb, nt, *ws)


def kernel(x, edge_index, edge_attr, node_type,
           fa_W, fa_b, fb_W, fb_b, fe_W1, fe_b1, fe_W2, fe_b2,
           fv_W1, fv_b1, fv_W2, fv_b2, m1_W1, m1_b1, m1_W2, m1_b2,
           m2_W1, m2_b1, m2_W2, m2_b2, m2_W3, m2_b3):
    f32 = jnp.float32
    Wh = fv_W1[:, :D]
    We = fv_W1[:, D:]
    s4 = edge_index[0].astype(jnp.int32).reshape(NH, NW, NCHUNK, C)
    r4 = edge_index[1].astype(jnp.int32).reshape(NH, NW, NCHUNK, C)
    c4 = jnp.concatenate([s4, r4], axis=-1)
    zeros = jnp.zeros((NP, D), f32)

    def b(v):
        return v.reshape(1, -1).astype(f32)

    def padw(w, rows, cols):
        return jnp.zeros((rows, cols), f32).at[:w.shape[0], :w.shape[1]].set(w)

    m1_W2p = padw(m1_W2, 8, D)
    m1_b2p = padw(m1_b2.reshape(1, -1), 1, 8)
    m2_W1p = padw(m2_W1, 16, DT)
    m2_b1p = padw(m2_b1.reshape(1, -1), 1, 16)
    m2_W2p = padw(m2_W2, 16, 16)
    m2_b2p = padw(m2_b2.reshape(1, -1), 1, 16)
    m2_W3p = padw(m2_W3, 16, 16)
    m2_b3p = padw(m2_b3.reshape(1, -1), 1, 16)

    h, hh = _init_call(x, fa_W, b(fa_b))
    eh = [edge_attr[:EH], edge_attr[EH:]]
    for step in range(3):
        ps = []
        for half in range(NH):
            hs, hhr = _gather_call(hh, c4[half])
            eh[half], msg = _edge_step(
                hs, hhr, eh[half], fb_W, b(fb_b),
                fe_W1, b(fe_b1), fe_W2, b(fe_b2),
                Wh, b(fv_b1), We, fv_W2, b(fv_b2), first=(step == 0))
            ps.append(_scatter_call(msg, r4[half], zeros))
        if step < 2:
            h, hh = _node_mid_call(h, ps[0], ps[1])
        else:
            force, g = _node_fin_call(
                h, ps[0], ps[1], node_type, m1_W1, b(m1_b1), m1_W2p, m1_b2p,
                m2_W1p, m2_b1p, m2_W2p, m2_b2p, m2_W3p, m2_b3p)
    return force[:, :3], g[:, :1]


# speedup vs baseline: 1.1762x; 1.1762x over previous
import jax
import jax.numpy as jnp
from jax import lax
from jax.experimental import pallas as pl
from jax.experimental.pallas import tpu as pltpu
from jax.experimental.pallas import tpu_sc as plsc

N = 10000
E = 320000
D = 128
DE = 16
DT = 16

NC = 2
NS = 16
NW = NC * NS
NH = 2
EH = E // NH
EW = EH // NW
C = 40
NCHUNK = EW // C
NP = 10240
ROWS_PT = NP // NS

BLK_E = 4000
BLK_N = 2000
BLK_T = 2048

_MESH = plsc.VectorSubcoreMesh(
    core_axis_name="c", subcore_axis_name="s", num_cores=NC, num_subcores=NS
)


def _sp(x):
    return jax.nn.softplus(x)


def _dot_t(x, w):
    return lax.dot_general(
        x, w, (((1,), (1,)), ((), ())), preferred_element_type=jnp.float32
    )


def _pack_bf16_pair(x):
    u = lax.bitcast_convert_type(x.astype(jnp.bfloat16), jnp.uint16)
    k = u.shape[1] // 2
    lo = u[:, :k].astype(jnp.uint32)
    hi = u[:, k:].astype(jnp.uint32)
    return lax.bitcast_convert_type(lo | (hi << 16), jnp.int32)


def _unpack_bf16_pair(w):
    f_lo = lax.bitcast_convert_type(lax.shift_left(w, 16), jnp.float32)
    f_hi = lax.bitcast_convert_type(
        jnp.bitwise_and(w, jnp.int32(-65536)), jnp.float32)
    return jnp.concatenate([f_lo, f_hi], axis=1)


def _pack2(x):
    hi = _pack_bf16_pair(x)
    lo = _pack_bf16_pair(x - _unpack_bf16_pair(hi))
    return jnp.concatenate([hi, lo], axis=1)


def _unpack2(w):
    k = w.shape[1] // 2
    return _unpack_bf16_pair(w[:, :k]) + _unpack_bf16_pair(w[:, k:])


def _gather_body(hh_hbm, c3_hbm, hs_hbm, hhr_hbm,
                 cidx, hsh, bg0, bg1, gs0, gs1, ws0, ws1):
    ci = lax.axis_index("c")
    si = lax.axis_index("s")
    wid = si * NC + ci
    pltpu.sync_copy(hh_hbm.at[pl.ds(si * ROWS_PT, ROWS_PT)],
                    hsh.at[pl.ds(si * ROWS_PT, ROWS_PT)])
    pltpu.sync_copy(c3_hbm.at[wid], cidx)
    plsc.subcore_barrier()

    def fire_gather(j, bg, sem):
        pltpu.async_copy(hsh.at[cidx.at[j]], bg, sem)

    def wait_gather(j, bg, sem):
        pltpu.make_async_copy(hsh.at[cidx.at[j]], bg, sem).wait()

    def fire_write(j, bg, sem):
        rows = wid * EW + j * C
        pltpu.async_copy(bg.at[pl.ds(0, C)], hs_hbm.at[pl.ds(rows, C)], sem)
        pltpu.async_copy(bg.at[pl.ds(C, C)], hhr_hbm.at[pl.ds(rows, C)], sem)

    def wait_write(j, bg, sem):
        rows = wid * EW + j * C
        pltpu.make_async_copy(
            bg.at[pl.ds(0, C)], hs_hbm.at[pl.ds(rows, C)], sem).wait()
        pltpu.make_async_copy(
            bg.at[pl.ds(C, C)], hhr_hbm.at[pl.ds(rows, C)], sem).wait()

    fire_gather(0, bg0, gs0)

    def body(t, carry):
        j0 = 2 * t
        fire_gather(j0 + 1, bg1, gs1)
        wait_gather(j0, bg0, gs0)
        fire_write(j0, bg0, ws0)
        wait_gather(j0 + 1, bg1, gs1)
        fire_write(j0 + 1, bg1, ws1)
        wait_write(j0, bg0, ws0)
        fire_gather(j0 + 2, bg0, gs0)
        wait_write(j0 + 1, bg1, ws1)
        return carry

    lax.fori_loop(0, (NCHUNK - 1) // 2, body, 0)
    last = NCHUNK - 1
    wait_gather(last, bg0, gs0)
    fire_write(last, bg0, ws0)
    wait_write(last, bg0, ws0)


_gather_call = pl.kernel(
    _gather_body,
    out_type=(
        jax.ShapeDtypeStruct((EH, D), jnp.int32),
        jax.ShapeDtypeStruct((EH, D), jnp.int32),
    ),
    mesh=_MESH,
    scratch_types=(
        pltpu.VMEM((NCHUNK, 2 * C), jnp.int32),
        pltpu.VMEM_SHARED((NP, D), jnp.int32),
        pltpu.VMEM((2 * C, D), jnp.int32),
        pltpu.VMEM((2 * C, D), jnp.int32),
        pltpu.SemaphoreType.DMA,
        pltpu.SemaphoreType.DMA,
        pltpu.SemaphoreType.DMA,
        pltpu.SemaphoreType.DMA,
    ),
)


def _scatter_body(msg_hbm, r3_hbm, z_hbm, p_hbm, ridx, b0, b1, acc, s0, s1):
    ci = lax.axis_index("c")
    si = lax.axis_index("s")
    wid = si * NC + ci
    pltpu.sync_copy(z_hbm.at[pl.ds(si * ROWS_PT, ROWS_PT)],
                    acc.at[pl.ds(si * ROWS_PT, ROWS_PT)])
    pltpu.sync_copy(r3_hbm.at[wid], ridx)
    plsc.subcore_barrier()

    def fire_load(t, buf, sem):
        rows = wid * EW + t * (2 * C)
        pltpu.async_copy(msg_hbm.at[pl.ds(rows, 2 * C)], buf, sem)

    def wait_load(t, buf, sem):
        rows = wid * EW + t * (2 * C)
        pltpu.make_async_copy(
            msg_hbm.at[pl.ds(rows, 2 * C)], buf, sem).wait()

    def add2(t, buf):
        pltpu.sync_copy(buf.at[pl.ds(0, C)], acc.at[ridx.at[2 * t]],
                        add=True)
        pltpu.sync_copy(buf.at[pl.ds(C, C)], acc.at[ridx.at[2 * t + 1]],
                        add=True)

    npair = NCHUNK // 2
    fire_load(0, b0, s0)

    def body(t, carry):
        t0 = 2 * t
        fire_load(t0 + 1, b1, s1)
        wait_load(t0, b0, s0)
        add2(t0, b0)
        fire_load(t0 + 2, b0, s0)
        wait_load(t0 + 1, b1, s1)
        add2(t0 + 1, b1)
        return carry

    lax.fori_loop(0, npair // 2 - 1, body, 0)
    fire_load(npair - 1, b1, s1)
    wait_load(npair - 2, b0, s0)
    add2(npair - 2, b0)
    wait_load(npair - 1, b1, s1)
    add2(npair - 1, b1)
    last = NCHUNK - 1
    rows = wid * EW + last * C
    pltpu.sync_copy(msg_hbm.at[pl.ds(rows, C)], b0.at[pl.ds(0, C)])
    pltpu.sync_copy(b0.at[pl.ds(0, C)], acc.at[ridx.at[last]], add=True)
    plsc.subcore_barrier()
    pltpu.sync_copy(acc.at[pl.ds(si * ROWS_PT, ROWS_PT)],
                    p_hbm.at[ci, pl.ds(si * ROWS_PT, ROWS_PT)])


_scatter_call = pl.kernel(
    _scatter_body,
    out_type=jax.ShapeDtypeStruct((NC, NP, D), jnp.float32),
    mesh=_MESH,
    scratch_types=(
        pltpu.VMEM((NCHUNK, C), jnp.int32),
        pltpu.VMEM((2 * C, D), jnp.float32),
        pltpu.VMEM((2 * C, D), jnp.float32),
        pltpu.VMEM_SHARED((NP, D), jnp.float32),
        pltpu.SemaphoreType.DMA,
        pltpu.SemaphoreType.DMA,
    ),
)


def _full_spec(a):
    nd = a.ndim
    return pl.BlockSpec(a.shape, lambda i, _nd=nd: (0,) * _nd)


def _init_body(x_ref, faW, fab, h_ref, hh_ref):
    h = _dot_t(x_ref[...], faW[...]) + fab[...]
    h_ref[...] = h
    hh_ref[...] = _pack2(h)


def _init_call(x, faW, fab):
    row = pl.BlockSpec((BLK_T, D), lambda i: (i, 0))
    return pl.pallas_call(
        _init_body,
        grid=(NP // BLK_T,),
        in_specs=[row] + [_full_spec(a) for a in (faW, fab)],
        out_specs=[row, row],
        out_shape=[
            jax.ShapeDtypeStruct((N, D), jnp.float32),
            jax.ShapeDtypeStruct((NP, D), jnp.int32),
        ],
    )(x, faW, fab)


def _edge_body_first(hs_ref, hhr_ref, ea_ref, fbW, fbb,
                     feW1, feb1, feW2, feb2, Wh, fvb1, We, fvW2, fvb2,
                     eout_ref, msg_ref):
    e_in = _dot_t(ea_ref[...], fbW[...]) + fbb[...]
    _edge_core(hs_ref, hhr_ref, e_in,
               feW1, feb1, feW2, feb2, Wh, fvb1, We, fvW2, fvb2,
               eout_ref, msg_ref)


def _edge_body_rest(hs_ref, hhr_ref, ein_ref,
                    feW1, feb1, feW2, feb2, Wh, fvb1, We, fvW2, fvb2,
                    eout_ref, msg_ref):
    _edge_core(hs_ref, hhr_ref, ein_ref[...],
               feW1, feb1, feW2, feb2, Wh, fvb1, We, fvW2, fvb2,
               eout_ref, msg_ref)


def _edge_core(hs_ref, hhr_ref, e_in,
               feW1, feb1, feW2, feb2, Wh, fvb1, We, fvW2, fvb2,
               eout_ref, msg_ref):
    hs = _unpack2(hs_ref[...])
    hrr = _unpack2(hhr_ref[...])
    hpr = _dot_t(hrr, Wh[...]) + fvb1[...]
    c2 = hs * hrr
    he = _sp(_dot_t(c2, feW1[...]) + feb1[...])
    e_new = _dot_t(he, feW2[...]) + feb2[...] + e_in
    hv = _sp(hpr + _dot_t(e_new, We[...]))
    msg = _dot_t(hv, fvW2[...]) + fvb2[...]
    eout_ref[...] = e_new
    msg_ref[...] = msg


def _edge_step(hs, hhr, ein, fbW, fbb,
               feW1, feb1, feW2, feb2, Wh, fvb1, We, fvW2, fvb2, first):
    row = pl.BlockSpec((BLK_E, D), lambda i: (i, 0))
    irow = pl.BlockSpec((BLK_E, D), lambda i: (i, 0))
    erow = pl.BlockSpec((BLK_E, DE), lambda i: (i, 0))
    if first:
        body = _edge_body_first
        winputs = (fbW, fbb, feW1, feb1, feW2, feb2, Wh, fvb1, We,
                   fvW2, fvb2)
    else:
        body = _edge_body_rest
        winputs = (feW1, feb1, feW2, feb2, Wh, fvb1, We, fvW2, fvb2)
    return pl.pallas_call(
        body,
        grid=(EH // BLK_E,),
        in_specs=[irow, irow, erow] + [_full_spec(a) for a in winputs],
        out_specs=[erow, row],
        out_shape=[
            jax.ShapeDtypeStruct((EH, DE), jnp.float32),
            jax.ShapeDtypeStruct((EH, D), jnp.float32),
        ],
    )(hs, hhr, ein, *winputs)


def _node_mid_body(h_ref, pa_ref, pb_ref, h_out, hh_out):
    hn = h_ref[...] + (pa_ref[0] + pa_ref[1]) + (pb_ref[0] + pb_ref[1])
    h_out[...] = hn
    hh_out[...] = _pack2(hn)


def _node_mid_call(h, pa, pb):
    row = pl.BlockSpec((BLK_T, D), lambda i: (i, 0))
    prow = pl.BlockSpec((NC, BLK_T, D), lambda i: (0, i, 0))
    return pl.pallas_call(
        _node_mid_body,
        grid=(NP // BLK_T,),
        in_specs=[row, prow, prow],
        out_specs=[row, row],
        out_shape=[
            jax.ShapeDtypeStruct((N, D), jnp.float32),
            jax.ShapeDtypeStruct((NP, D), jnp.int32),
        ],
    )(h, pa, pb)


def _node_fin_body(h_ref, pa_ref, pb_ref, nt_ref,
                   m1W1, m1b1, m1W2, m1b2,
                   m2W1, m2b1, m2W2, m2b2, m2W3, m2b3,
                   force_ref, g_ref):
    hn = h_ref[...] + (pa_ref[0] + pa_ref[1]) + (pb_ref[0] + pb_ref[1])
    t = _sp(_dot_t(hn, m1W1[...]) + m1b1[...])
    force_ref[...] = _dot_t(t, m1W2[...]) + m1b2[...]
    g = _sp(_dot_t(nt_ref[...], m2W1[...]) + m2b1[...])
    g = _sp(_dot_t(g, m2W2[...]) + m2b2[...])
    g_ref[...] = _sp(_dot_t(g, m2W3[...]) + m2b3[...])


def _node_fin_call(h, pa, pb, nt, m1W1, m1b1, m1W2, m1b2,
                   m2W1, m2b1, m2W2, m2b2, m2W3, m2b3):
    row = pl.BlockSpec((BLK_N, D), lambda i: (i, 0))
    prow = pl.BlockSpec((NC, BLK_N, D), lambda i: (0, i, 0))
    ntrow = pl.BlockSpec((BLK_N, DT), lambda i: (i, 0))
    ws = (m1W1, m1b1, m1W2, m1b2, m2W1, m2b1, m2W2, m2b2, m2W3, m2b3)
    return pl.pallas_call(
        _node_fin_body,
        grid=(N // BLK_N,),
        in_specs=[row, prow, prow, ntrow] + [_full_spec(a) for a in ws],
        out_specs=[
            pl.BlockSpec((BLK_N, 8), lambda i: (i, 0)),
            pl.BlockSpec((BLK_N, 16), lambda i: (i, 0)),
        ],
        out_shape=[
            jax.ShapeDtypeStruct((N, 8), jnp.float32),
            jax.ShapeDtypeStruct((N, 16), jnp.float32),
        ],
    )(h, pa, pb, nt, *ws)


def kernel(x, edge_index, edge_attr, node_type,
           fa_W, fa_b, fb_W, fb_b, fe_W1, fe_b1, fe_W2, fe_b2,
           fv_W1, fv_b1, fv_W2, fv_b2, m1_W1, m1_b1, m1_W2, m1_b2,
           m2_W1, m2_b1, m2_W2, m2_b2, m2_W3, m2_b3):
    f32 = jnp.float32
    Wh = fv_W1[:, :D]
    We = fv_W1[:, D:]
    s4 = edge_index[0].astype(jnp.int32).reshape(NH, NW, NCHUNK, C)
    r4 = edge_index[1].astype(jnp.int32).reshape(NH, NW, NCHUNK, C)
    c4 = jnp.concatenate([s4, r4], axis=-1)
    zeros = jnp.zeros((NP, D), f32)

    def b(v):
        return v.reshape(1, -1).astype(f32)

    def padw(w, rows, cols):
        return jnp.zeros((rows, cols), f32).at[:w.shape[0], :w.shape[1]].set(w)

    m1_W2p = padw(m1_W2, 8, D)
    m1_b2p = padw(m1_b2.reshape(1, -1), 1, 8)
    m2_W1p = padw(m2_W1, 16, DT)
    m2_b1p = padw(m2_b1.reshape(1, -1), 1, 16)
    m2_W2p = padw(m2_W2, 16, 16)
    m2_b2p = padw(m2_b2.reshape(1, -1), 1, 16)
    m2_W3p = padw(m2_W3, 16, 16)
    m2_b3p = padw(m2_b3.reshape(1, -1), 1, 16)

    h, hh = _init_call(x, fa_W, b(fa_b))
    eh = [edge_attr[:EH], edge_attr[EH:]]
    for step in range(3):
        ps = []
        for half in range(NH):
            hs, hhr = _gather_call(hh, c4[half])
            eh[half], msg = _edge_step(
                hs, hhr, eh[half], fb_W, b(fb_b),
                fe_W1, b(fe_b1), fe_W2, b(fe_b2),
                Wh, b(fv_b1), We, fv_W2, b(fv_b2), first=(step == 0))
            ps.append(_scatter_call(msg, r4[half], zeros))
        if step < 2:
            h, hh = _node_mid_call(h, ps[0], ps[1])
        else:
            force, g = _node_fin_call(
                h, ps[0], ps[1], node_type, m1_W1, b(m1_b1), m1_W2p, m1_b2p,
                m2_W1p, m2_b1p, m2_W2p, m2_b2p, m2_W3p, m2_b3p)
    return force[:, :3], g[:, :1]
